# Initial kernel scaffold; baseline (speedup 1.0000x reference)
import jax
import jax.numpy as jnp
from jax.experimental import pallas as pl


def kernel(edge_index, E, W1, b1, W2, b2, W3, b3):
    raise NotImplementedError("write your pallas kernel here")


# speedup vs baseline: 6.4630x; 6.4630x over previous
import functools

import jax
import jax.numpy as jnp
from jax import lax
from jax.experimental import pallas as pl
from jax.experimental.pallas import tpu as pltpu
from jax.experimental.pallas import tpu_sc as plsc

N = 50000
D = 64
EDGES = 800000

CH = 128
NSUB = 16
NCORE = 2
EPAD = 819200
NCHUNKS = EPAD // CH
CPT = NCHUNKS // NSUB
HALF = 25000
ACC_CHUNKS_PER_TILE = 13
ACC_R = NSUB * ACC_CHUNKS_PER_TILE * CH
DUMMY = HALF
DEG_R = 51200
DEG_CPT = NCHUNKS // (NSUB * NCORE)

_mesh = plsc.VectorSubcoreMesh(core_axis_name="c", subcore_axis_name="s")


def _fill(ref, rows, cols, value):
    vec = jnp.full((16,), value, jnp.float32)

    def body(i, _):
        for k in range(cols // 16):
            ref[i, pl.ds(k * 16, 16)] = vec
        return 0

    lax.fori_loop(0, rows, body, 0)


def _fill1d(ref, n, value):
    vec = jnp.full((16,), value, jnp.float32)

    def body(i, _):
        ref[pl.ds(i * 16, 16)] = vec
        return 0

    lax.fori_loop(0, n // 16, body, 0)


@functools.partial(
    pl.kernel,
    out_type=jax.ShapeDtypeStruct((NCORE * DEG_R,), jnp.float32),
    mesh=_mesh,
    scratch_types=[
        pltpu.VMEM((CH,), jnp.int32),
        pltpu.VMEM((CH,), jnp.float32),
        pltpu.VMEM((1600,), jnp.float32),
        pltpu.VMEM_SHARED((DEG_R,), jnp.float32),
    ],
)
def _sc_deg(col_hbm, out_hbm, col_v, ones_v, stage_v, deg_sh):
    c = lax.axis_index("c")
    s = lax.axis_index("s")
    _fill1d(ones_v, CH, 1.0)
    _fill1d(stage_v, 1600, 0.0)
    for t in range(2):
        pltpu.sync_copy(stage_v, deg_sh.at[pl.ds(s * 3200 + t * 1600, 1600)])
    plsc.subcore_barrier()
    base_edge = (c * (NSUB * DEG_CPT) + s * DEG_CPT) * CH

    def body(j, _):
        pltpu.sync_copy(col_hbm.at[pl.ds(base_edge + j * CH, CH)], col_v)
        pltpu.sync_copy(ones_v, deg_sh.at[col_v], add=True)
        return 0

    lax.fori_loop(0, DEG_CPT, body, 0)
    plsc.subcore_barrier()
    for t in range(2):
        off = s * 3200 + t * 1600
        pltpu.sync_copy(deg_sh.at[pl.ds(off, 1600)], stage_v)
        pltpu.sync_copy(stage_v, out_hbm.at[pl.ds(c * DEG_R + off, 1600)])


@functools.partial(
    pl.kernel,
    out_type=jax.ShapeDtypeStruct((NCORE, ACC_R, D), jnp.float32),
    mesh=_mesh,
    scratch_types=[
        pltpu.VMEM((CH,), jnp.int32),
        pltpu.VMEM((CH,), jnp.int32),
        pltpu.VMEM((CH, D), jnp.float32),
        pltpu.VMEM((CH, D), jnp.float32),
        pltpu.VMEM_SHARED((ACC_R, D), jnp.float32),
        pltpu.SemaphoreType.DMA,
    ],
    compiler_params=pltpu.CompilerParams(use_tc_tiling_on_sc=False),
)
def _sc_agg(row_hbm, col_hbm, z_hbm, out_hbm, row_v, col_v, rows_v, zero_v,
            acc_sh, sem):
    c = lax.axis_index("c")
    s = lax.axis_index("s")
    base_node = c * HALF
    _fill(zero_v, CH, D, 0.0)
    for q in range(ACC_CHUNKS_PER_TILE):
        pltpu.sync_copy(
            zero_v, acc_sh.at[pl.ds((s * ACC_CHUNKS_PER_TILE + q) * CH, CH)])
    plsc.subcore_barrier()

    def body(j, _):
        g = (s * CPT + j) * CH
        pltpu.sync_copy(row_hbm.at[pl.ds(g, CH)], row_v)
        pltpu.sync_copy(col_hbm.at[pl.ds(g, CH)], col_v)

        def cbody(k, _):
            cv = col_v[pl.ds(k * 16, 16)]
            lc = cv - base_node
            ok = (lc >= 0) & (lc < HALF)
            col_v[pl.ds(k * 16, 16)] = jnp.where(ok, lc, DUMMY)
            return 0

        lax.fori_loop(0, CH // 16, cbody, 0)
        pltpu.async_copy(z_hbm.at[row_v], rows_v, sem).wait()
        pltpu.sync_copy(rows_v, acc_sh.at[col_v], add=True)
        return 0

    lax.fori_loop(0, CPT, body, 0)
    plsc.subcore_barrier()
    for q in range(ACC_CHUNKS_PER_TILE):
        st = (s * ACC_CHUNKS_PER_TILE + q) * CH
        pltpu.sync_copy(acc_sh.at[pl.ds(st, CH)], rows_v)
        pltpu.sync_copy(rows_v, out_hbm.at[c, pl.ds(st, CH)])


BLK = 1000
GRID = N // BLK


def _tc_pre_body(e_ref, d0_ref, d1_ref, wt_ref, b_ref, dinv_ref, z_ref):
    dinv = lax.rsqrt(d0_ref[...] + d1_ref[...] + 1.0)
    y = jnp.dot(e_ref[...], wt_ref[...], preferred_element_type=jnp.float32)
    dinv_ref[...] = dinv
    z_ref[...] = dinv * (y + b_ref[...])


def _tc_mid_body(acc_ref, z_ref, dinv_ref, wt_ref, b_ref, e_ref, zn_ref):
    o = dinv_ref[...] * (acc_ref[...] + z_ref[...])
    o = jnp.where(o >= 0, o, 0.01 * o)
    nrm = jnp.sqrt(jnp.sum(o * o, axis=1, keepdims=True))
    o = o / jnp.maximum(nrm, 1e-12)
    e_ref[...] = o
    zn_ref[...] = dinv_ref[...] * (
        jnp.dot(o, wt_ref[...], preferred_element_type=jnp.float32) + b_ref[...])


def _tc_post_body(acc_ref, z_ref, dinv_ref, e_ref):
    o = dinv_ref[...] * (acc_ref[...] + z_ref[...])
    o = jnp.where(o >= 0, o, 0.01 * o)
    nrm = jnp.sqrt(jnp.sum(o * o, axis=1, keepdims=True))
    e_ref[...] = o / jnp.maximum(nrm, 1e-12)


_row_spec = pl.BlockSpec((BLK, D), lambda i: (i, 0))
_col1_spec = pl.BlockSpec((BLK, 1), lambda i: (i, 0))
_w_spec = pl.BlockSpec((D, D), lambda i: (0, 0))
_b_spec = pl.BlockSpec((1, D), lambda i: (0, 0))

_tc_pre = pl.pallas_call(
    _tc_pre_body,
    grid=(GRID,),
    in_specs=[_row_spec, _col1_spec, _col1_spec, _w_spec, _b_spec],
    out_specs=[_col1_spec, _row_spec],
    out_shape=[
        jax.ShapeDtypeStruct((N, 1), jnp.float32),
        jax.ShapeDtypeStruct((N, D), jnp.float32),
    ],
)

_tc_mid = pl.pallas_call(
    _tc_mid_body,
    grid=(GRID,),
    in_specs=[_row_spec, _row_spec, _col1_spec, _w_spec, _b_spec],
    out_specs=[_row_spec, _row_spec],
    out_shape=[
        jax.ShapeDtypeStruct((N, D), jnp.float32),
        jax.ShapeDtypeStruct((N, D), jnp.float32),
    ],
)

_tc_post = pl.pallas_call(
    _tc_post_body,
    grid=(GRID,),
    in_specs=[_row_spec, _row_spec, _col1_spec],
    out_specs=_row_spec,
    out_shape=jax.ShapeDtypeStruct((N, D), jnp.float32),
)


def kernel(edge_index, E, W1, b1, W2, b2, W3, b3):
    ei = edge_index.astype(jnp.int32)
    pad = EPAD - EDGES
    row = jnp.concatenate([ei[0], jnp.zeros((pad,), jnp.int32)])
    col = jnp.concatenate([ei[1], jnp.full((pad,), N, jnp.int32)])

    deg2 = _sc_deg(col)
    d0 = deg2[:N, None]
    d1 = deg2[DEG_R:DEG_R + N, None]

    dinv, z1 = _tc_pre(E, d0, d1, W1.T, b1[None, :])

    def agg(z):
        a = _sc_agg(row, col, z)
        return jnp.concatenate([a[0, :HALF], a[1, :HALF]], axis=0)

    E1, z2 = _tc_mid(agg(z1), z1, dinv, W2.T, b2[None, :])
    E2, z3 = _tc_mid(agg(z2), z2, dinv, W3.T, b3[None, :])
    E3 = _tc_post(agg(z3), z3, dinv)
    return jnp.concatenate([E, E1, E2, E3], axis=1)
